# baseline (speedup 1.0000x reference)
import functools

import jax
import jax.numpy as jnp
from jax import lax
from jax.experimental import pallas as pl
from jax.experimental.pallas import tpu as pltpu
from jax.experimental.pallas import tpu_sc as plsc

N_NODES = 100000
N_EDGES = 6400000
D_FEAT = 5
DP = 8

NC = 2
NS = 16
NW = NC * NS

B = 80
RO = 100
ROWS = N_EDGES // B
NO = ROWS // (NW * RO)
K = 5
A = 3
NG = RO // K
NP = 100096
RPT = NP // NS


def _sc_body(edge_hbm, f_hbm, zeros_hbm, out_hbm,
             f_sh, h_sh, src_buf, dst_buf,
             rows0, rows1, rows2, rows3, rows4,
             gsem, ssem):
    c = lax.axis_index("c")
    s = lax.axis_index("s")
    wid = c * NS + s
    r0 = s * RPT
    rows_v = (rows0, rows1, rows2, rows3, rows4)
    pltpu.sync_copy(f_hbm.at[pl.ds(r0, RPT)], f_sh.at[pl.ds(r0, RPT)])
    pltpu.sync_copy(zeros_hbm.at[pl.ds(r0, RPT)], h_sh.at[pl.ds(r0, RPT)])
    plsc.subcore_barrier()

    def gather(j, k):
        return pltpu.make_async_copy(
            f_sh.at[src_buf.at[pl.ds(j * B, B)]], rows_v[k], gsem.at[k])

    def scat(j, k):
        return pltpu.make_async_copy(
            rows_v[k], h_sh.at[dst_buf.at[pl.ds(j * B, B)]], ssem.at[k])

    def outer(o, _):
        off = (wid * NO + o) * (RO * B)
        pltpu.sync_copy(edge_hbm.at[0, pl.ds(off, RO * B)], src_buf)
        pltpu.sync_copy(edge_hbm.at[1, pl.ds(off, RO * B)], dst_buf)
        for k in range(A):
            gather(k, k).start()

        for k in range(K):
            ka = (k + A) % K
            if k >= K - A:
                scat(k - (K - A), ka).wait()
            gather(k + A, ka).start()
            gather(k, k).wait()
            scat(k, k).start(add=True)

        def group(g, _):
            for k in range(K):
                j = g * K + k
                ka = (k + A) % K
                scat(j - (K - A), ka).wait()
                gather(j + A, ka).start()
                gather(j, k).wait()
                scat(j, k).start(add=True)
            return 0

        lax.fori_loop(1, NG - 1, group, 0)
        for k in range(K):
            j = (NG - 1) * K + k
            ka = (k + A) % K
            if k + A < K:
                scat(j - (K - A), ka).wait()
                gather(j + A, ka).start()
            gather(j, k).wait()
            scat(j, k).start(add=True)
        for k in range(K):
            scat(0, k).wait()
        return 0

    lax.fori_loop(0, NO, outer, 0)
    plsc.subcore_barrier()
    pltpu.sync_copy(h_sh.at[pl.ds(r0, RPT)], out_hbm.at[c, pl.ds(r0, RPT)])


_sc_call = functools.partial(
    pl.kernel,
    out_type=jax.ShapeDtypeStruct((NC, NP, DP), jnp.float32),
    mesh=plsc.VectorSubcoreMesh(core_axis_name="c", subcore_axis_name="s"),
    compiler_params=pltpu.CompilerParams(use_tc_tiling_on_sc=False),
    scratch_types=[
        pltpu.VMEM_SHARED((NP, DP), jnp.float32),
        pltpu.VMEM_SHARED((NP, DP), jnp.float32),
        pltpu.VMEM((RO * B,), jnp.int32),
        pltpu.VMEM((RO * B,), jnp.int32),
        pltpu.VMEM((B, DP), jnp.float32),
        pltpu.VMEM((B, DP), jnp.float32),
        pltpu.VMEM((B, DP), jnp.float32),
        pltpu.VMEM((B, DP), jnp.float32),
        pltpu.VMEM((B, DP), jnp.float32),
        pltpu.SemaphoreType.DMA((K,)),
        pltpu.SemaphoreType.DMA((K,)),
    ],
)(_sc_body)


def _tc_sum_body(p_ref, o_ref):
    o_ref[...] = p_ref[0] + p_ref[1]


def kernel(edge_index, f):
    f_pad = jnp.pad(f, ((0, NP - N_NODES), (0, DP - D_FEAT)))
    zeros = jnp.zeros((NP, DP), jnp.float32)
    partials = _sc_call(edge_index, f_pad, zeros)
    flat = partials.reshape(NC, (NP * DP) // 128, 128)
    summed = pl.pallas_call(
        _tc_sum_body,
        out_shape=jax.ShapeDtypeStruct(((NP * DP) // 128, 128), jnp.float32),
    )(flat)
    return summed.reshape(NP, DP)[:N_NODES, :D_FEAT]


# speedup vs baseline: 1.0325x; 1.0325x over previous
import functools

import jax
import jax.numpy as jnp
from jax import lax
from jax.experimental import pallas as pl
from jax.experimental.pallas import tpu as pltpu
from jax.experimental.pallas import tpu_sc as plsc

N_NODES = 100000
N_EDGES = 6400000
D_FEAT = 5
DP = 8

NC = 2
NS = 16
NW = NC * NS

B = 80
RO = 100
ROWS = N_EDGES // B
NO = ROWS // (NW * RO)
K = 5
A = 2
NG = RO // K
NP = 100096
RPT = NP // NS


def _sc_body(edge_hbm, f_hbm, zeros_hbm, out_hbm,
             f_sh, h_sh, src_buf, dst_buf,
             rows0, rows1, rows2, rows3, rows4,
             gsem, ssem):
    c = lax.axis_index("c")
    s = lax.axis_index("s")
    wid = c * NS + s
    r0 = s * RPT
    rows_v = (rows0, rows1, rows2, rows3, rows4)
    pltpu.sync_copy(f_hbm.at[pl.ds(r0, RPT)], f_sh.at[pl.ds(r0, RPT)])
    pltpu.sync_copy(zeros_hbm.at[pl.ds(r0, RPT)], h_sh.at[pl.ds(r0, RPT)])
    plsc.subcore_barrier()

    def gather(j, k):
        return pltpu.make_async_copy(
            f_sh.at[src_buf.at[pl.ds(j * B, B)]], rows_v[k], gsem.at[k])

    def scat(j, k):
        return pltpu.make_async_copy(
            rows_v[k], h_sh.at[dst_buf.at[pl.ds(j * B, B)]], ssem.at[k])

    def outer(o, _):
        off = (wid * NO + o) * (RO * B)
        pltpu.sync_copy(edge_hbm.at[0, pl.ds(off, RO * B)], src_buf)
        pltpu.sync_copy(edge_hbm.at[1, pl.ds(off, RO * B)], dst_buf)
        for k in range(A):
            gather(k, k).start()

        for k in range(K):
            ka = (k + A) % K
            if k >= K - A:
                scat(k - (K - A), ka).wait()
            gather(k + A, ka).start()
            gather(k, k).wait()
            scat(k, k).start(add=True)

        def group(g, _):
            for k in range(K):
                j = g * K + k
                ka = (k + A) % K
                scat(j - (K - A), ka).wait()
                gather(j + A, ka).start()
                gather(j, k).wait()
                scat(j, k).start(add=True)
            return 0

        lax.fori_loop(1, NG - 1, group, 0)
        for k in range(K):
            j = (NG - 1) * K + k
            ka = (k + A) % K
            if k + A < K:
                scat(j - (K - A), ka).wait()
                gather(j + A, ka).start()
            gather(j, k).wait()
            scat(j, k).start(add=True)
        for k in range(K):
            scat(0, k).wait()
        return 0

    lax.fori_loop(0, NO, outer, 0)
    plsc.subcore_barrier()
    pltpu.sync_copy(h_sh.at[pl.ds(r0, RPT)], out_hbm.at[c, pl.ds(r0, RPT)])


def _make_sc_call():
  return functools.partial(
      pl.kernel,
      out_type=jax.ShapeDtypeStruct((NC, NP, DP), jnp.float32),
      mesh=plsc.VectorSubcoreMesh(core_axis_name="c", subcore_axis_name="s",
                                  num_cores=NC, num_subcores=NS),
      compiler_params=pltpu.CompilerParams(use_tc_tiling_on_sc=False),
      scratch_types=[
          pltpu.VMEM_SHARED((NP, DP), jnp.float32),
          pltpu.VMEM_SHARED((NP, DP), jnp.float32),
          pltpu.VMEM((RO * B,), jnp.int32),
          pltpu.VMEM((RO * B,), jnp.int32),
          pltpu.VMEM((B, DP), jnp.float32),
          pltpu.VMEM((B, DP), jnp.float32),
          pltpu.VMEM((B, DP), jnp.float32),
          pltpu.VMEM((B, DP), jnp.float32),
          pltpu.VMEM((B, DP), jnp.float32),
          pltpu.SemaphoreType.DMA((K,)),
          pltpu.SemaphoreType.DMA((K,)),
      ],
    )(_sc_body)


def _tc_sum_body(p_ref, o_ref):
    o_ref[...] = p_ref[0] + p_ref[1]


def kernel(edge_index, f):
    f_pad = jnp.pad(f, ((0, NP - N_NODES), (0, DP - D_FEAT)))
    zeros = jnp.zeros((NP, DP), jnp.float32)
    partials = _make_sc_call()(edge_index, f_pad, zeros)
    flat = partials.reshape(NC, (NP * DP) // 128, 128)
    summed = pl.pallas_call(
        _tc_sum_body,
        out_shape=jax.ShapeDtypeStruct(((NP * DP) // 128, 128), jnp.float32),
    )(flat)
    return summed.reshape(NP, DP)[:N_NODES, :D_FEAT]
